# Initial kernel scaffold; baseline (speedup 1.0000x reference)
import jax
import jax.numpy as jnp
from jax.experimental import pallas as pl


def kernel(x, pos, batch, sa1, sa2, sa3, head):
    raise NotImplementedError("write your pallas kernel here")


# speedup vs baseline: 1.2200x; 1.2200x over previous
import functools

import jax
import jax.numpy as jnp
from jax.experimental import pallas as pl

B = 16
N = 2048
MAX_NB = 64
NEG_INF = float("-inf")


def _fps_body(px_ref, py_ref, pz_ref, q1x_ref, q1y_ref, q1z_ref,
              q2x_ref, q2y_ref, q2z_ref):
    def run_fps(px, py, pz, n_s):
        n = px.shape[1]
        lane = jax.lax.broadcasted_iota(jnp.int32, (B, n), 1)
        qlane = jax.lax.broadcasted_iota(jnp.int32, (B, n_s), 1)
        dx = px - px[:, 0:1]
        dy = py - py[:, 0:1]
        dz = pz - pz[:, 0:1]
        d = (dx * dx + dy * dy) + dz * dz
        qx0 = jnp.broadcast_to(px[:, 0:1], (B, 1))

        def step(i, carry):
            d, ax, ay, az = carry
            m = jnp.max(d, axis=1, keepdims=True)
            cand = jnp.where(d == m, lane, n)
            nxt = jnp.min(cand, axis=1, keepdims=True)
            oh = (lane == nxt)
            qx = jnp.sum(jnp.where(oh, px, 0.0), axis=1, keepdims=True)
            qy = jnp.sum(jnp.where(oh, py, 0.0), axis=1, keepdims=True)
            qz = jnp.sum(jnp.where(oh, pz, 0.0), axis=1, keepdims=True)
            ddx = px - qx
            ddy = py - qy
            ddz = pz - qz
            dn = (ddx * ddx + ddy * ddy) + ddz * ddz
            d = jnp.minimum(d, dn)
            ohq = (qlane == i)
            ax = jnp.where(ohq, qx, ax)
            ay = jnp.where(ohq, qy, ay)
            az = jnp.where(ohq, qz, az)
            return d, ax, ay, az

        ax0 = jnp.broadcast_to(px[:, 0:1], (B, n_s))
        ay0 = jnp.broadcast_to(py[:, 0:1], (B, n_s))
        az0 = jnp.broadcast_to(pz[:, 0:1], (B, n_s))
        _, ax, ay, az = jax.lax.fori_loop(1, n_s, step, (d, ax0, ay0, az0))
        return ax, ay, az

    q1x, q1y, q1z = run_fps(px_ref[...], py_ref[...], pz_ref[...], N // 2)
    q1x_ref[...] = q1x
    q1y_ref[...] = q1y
    q1z_ref[...] = q1z
    q2x, q2y, q2z = run_fps(q1x, q1y, q1z, N // 8)
    q2x_ref[...] = q2x
    q2y_ref[...] = q2y
    q2z_ref[...] = q2z


def _fps_call(px, py, pz):
    M1, M2 = N // 2, N // 8
    f32 = jnp.float32
    outs = [jax.ShapeDtypeStruct((B, M1), f32)] * 3 + \
           [jax.ShapeDtypeStruct((B, M2), f32)] * 3
    return pl.pallas_call(_fps_body, out_shape=outs)(px, py, pz)


def _mlp_max_body(msg_ref, mask_ref, w1_ref, b1_ref, w2_ref, b2_ref,
                  w3_ref, b3_ref, out_ref, *, s_chunk, m):
    c = pl.program_id(1)
    nc = pl.num_programs(1)
    x = msg_ref[0]
    h = jnp.dot(x, w1_ref[...], preferred_element_type=jnp.float32) + b1_ref[...]
    h = jax.nn.relu(h)
    h = jnp.dot(h, w2_ref[...], preferred_element_type=jnp.float32) + b2_ref[...]
    h = jax.nn.relu(h)
    h = jnp.dot(h, w3_ref[...], preferred_element_type=jnp.float32) + b3_ref[...]
    mask = mask_ref[0, 0]
    h = jnp.where(mask[:, None] > 0.5, h, NEG_INF)
    part = h[0 * m:(0 + 1) * m]
    for s in range(1, s_chunk):
        part = jnp.maximum(part, h[s * m:(s + 1) * m])

    @pl.when(c == 0)
    def _():
        out_ref[0] = part

    @pl.when(c > 0)
    def _():
        out_ref[0] = jnp.maximum(out_ref[0], part)

    @pl.when(c == nc - 1)
    def _():
        o = out_ref[0]
        out_ref[0] = jnp.where(o == NEG_INF, 0.0, o)


def _mlp_max_call(msg, mask, params, m, s_chunk):
    (w1, b1), (w2, b2), (w3, b3) = params
    S = MAX_NB
    nc = S // s_chunk
    cin = msg.shape[-1]
    cout = w3.shape[1]
    body = functools.partial(_mlp_max_body, s_chunk=s_chunk, m=m)
    grid = (B, nc)
    mask3 = mask.reshape(B * nc, 1, s_chunk * m)
    return pl.pallas_call(
        body,
        grid=grid,
        in_specs=[
            pl.BlockSpec((1, s_chunk * m, cin), lambda b, c: (b, c, 0)),
            pl.BlockSpec((1, 1, s_chunk * m), lambda b, c: (b * nc + c, 0, 0)),
            pl.BlockSpec(w1.shape, lambda b, c: (0, 0)),
            pl.BlockSpec(b1.shape, lambda b, c: (0,)),
            pl.BlockSpec(w2.shape, lambda b, c: (0, 0)),
            pl.BlockSpec(b2.shape, lambda b, c: (0,)),
            pl.BlockSpec(w3.shape, lambda b, c: (0, 0)),
            pl.BlockSpec(b3.shape, lambda b, c: (0,)),
        ],
        out_specs=pl.BlockSpec((1, m, cout), lambda b, c: (b, 0, 0)),
        out_shape=jax.ShapeDtypeStruct((B, m, cout), jnp.float32),
    )(msg, mask3, w1, b1, w2, b2, w3, b3)


def _tail_body(x_ref, w31_ref, b31_ref, w32_ref, b32_ref, w33_ref, b33_ref,
               h1_ref, c1_ref, h2_ref, c2_ref, h3_ref, c3_ref, out_ref, *, m):
    x = x_ref[...]
    h = jnp.dot(x, w31_ref[...], preferred_element_type=jnp.float32) + b31_ref[...]
    h = jax.nn.relu(h)
    h = jnp.dot(h, w32_ref[...], preferred_element_type=jnp.float32) + b32_ref[...]
    h = jax.nn.relu(h)
    h = jnp.dot(h, w33_ref[...], preferred_element_type=jnp.float32) + b33_ref[...]
    g = jnp.concatenate(
        [jnp.max(h[b * m:(b + 1) * m], axis=0, keepdims=True) for b in range(B)],
        axis=0)
    g = jnp.dot(g, h1_ref[...], preferred_element_type=jnp.float32) + c1_ref[...]
    g = jax.nn.relu(g)
    g = jnp.dot(g, h2_ref[...], preferred_element_type=jnp.float32) + c2_ref[...]
    g = jax.nn.relu(g)
    g = jnp.dot(g, h3_ref[...], preferred_element_type=jnp.float32) + c3_ref[...]
    out_ref[...] = g


def _tail_call(x, sa3, head, m):
    (w31, b31), (w32, b32), (w33, b33) = sa3
    (h1, c1), (h2, c2), (h3, c3) = head
    body = functools.partial(_tail_body, m=m)
    return pl.pallas_call(
        body,
        out_shape=jax.ShapeDtypeStruct((B, h3.shape[1]), jnp.float32),
    )(x, w31, b31, w32, b32, w33, b33, h1, c1, h2, c2, h3, c3)


def _radius_topk(qx, qy, qz, px, py, pz, r):
    q = jnp.stack([qx, qy, qz], axis=-1)
    p = jnp.stack([px, py, pz], axis=-1)

    def one(qc, pc):
        d2 = (jnp.sum(qc ** 2, 1)[:, None] + jnp.sum(pc ** 2, 1)[None, :]
              - 2.0 * (qc @ pc.T))
        valid = d2 <= r * r
        score = jnp.where(valid, -d2, -jnp.inf)
        vals, nbr = jax.lax.top_k(score, MAX_NB)
        return nbr, vals > -jnp.inf

    return jax.vmap(one)(q, p)


def _build_msg(feat, px, py, pz, qx, qy, qz, nbr, mask):
    Bq, M, S = nbr.shape
    f_g = jnp.take_along_axis(feat[:, None], nbr[..., None], axis=2)
    p = jnp.stack([px, py, pz], axis=-1)
    p_g = jnp.take_along_axis(p[:, None], nbr[..., None], axis=2)
    q = jnp.stack([qx, qy, qz], axis=-1)
    rel = p_g - q[:, :, None, :]
    msg = jnp.concatenate([f_g, rel], axis=-1)
    msg = msg.transpose(0, 2, 1, 3).reshape(Bq, S * M, -1)
    maskf = mask.astype(jnp.float32).transpose(0, 2, 1).reshape(Bq, S * M)
    return msg, maskf


def kernel(x, pos, batch, sa1, sa2, sa3, head):
    del batch
    posb = pos.reshape(B, N, 3)
    xb = x.reshape(B, N, 3)
    px, py, pz = posb[..., 0], posb[..., 1], posb[..., 2]

    q1x, q1y, q1z, q2x, q2y, q2z = _fps_call(px, py, pz)
    M1, M2 = N // 2, N // 8

    nbr1, mask1 = _radius_topk(q1x, q1y, q1z, px, py, pz, 0.2)
    msg1, maskf1 = _build_msg(xb, px, py, pz, q1x, q1y, q1z, nbr1, mask1)
    x1 = _mlp_max_call(msg1, maskf1, sa1, M1, s_chunk=8)

    nbr2, mask2 = _radius_topk(q2x, q2y, q2z, q1x, q1y, q1z, 0.4)
    msg2, maskf2 = _build_msg(x1, q1x, q1y, q1z, q2x, q2y, q2z, nbr2, mask2)
    x2 = _mlp_max_call(msg2, maskf2, sa2, M2, s_chunk=16)

    q2 = jnp.stack([q2x, q2y, q2z], axis=-1)
    tail_in = jnp.concatenate([x2, q2], axis=-1).reshape(B * M2, 259)
    return _tail_call(tail_in, sa3, head, M2)
